# baseline (speedup 1.0000x reference)
import functools

import jax
import jax.numpy as jnp
from jax import lax
from jax.experimental import pallas as pl
from jax.experimental.pallas import tpu as pltpu
from jax.experimental.pallas import tpu_sc as plsc

F32 = jnp.float32

NV = 50000
NCK = 25000
NE = 800000
B = 16
WP = 32
ITERS = 5

NC = 2
NS = 16
NW = NC * NS
CHUNK = 128
CPT = 196
EPAD = NW * CPT * CHUNK
EROWS = EPAD * B // 1024

S_CHK = 25024
S_VAR = 50048

_mesh = plsc.VectorSubcoreMesh(
    core_axis_name="c", subcore_axis_name="s", num_cores=NC, num_subcores=NS)
_sc_params = pltpu.CompilerParams(use_tc_tiling_on_sc=False)


def _wid():
    return lax.axis_index("s") * NC + lax.axis_index("c")


def _scatter_body(nseg, wd, vb, vals, ids, zeros, out,
                  idv, buf, table, sl0, sl1, ss0, ss1):
    rs = nseg // NS
    njo = CPT // vb
    c = lax.axis_index("c")
    s = lax.axis_index("s")
    w = _wid()
    pltpu.sync_copy(zeros.at[s], table.at[pl.ds(s * rs, rs)])
    plsc.subcore_barrier()

    sems = (sl0, sl1)

    def load(jo, par):
        pltpu.async_copy(ids.at[w, pl.ds(jo * vb, vb)], idv.at[par], sems[par])
        pltpu.async_copy(vals.at[w, jo], buf.at[par], sems[par])

    def load_wait(jo, par):
        pltpu.make_async_copy(
            ids.at[w, pl.ds(jo * vb, vb)], idv.at[par], sems[par]).wait()
        pltpu.make_async_copy(vals.at[w, jo], buf.at[par], sems[par]).wait()

    load(0, 0)
    load(1, 1)

    def body(jo2, _):
        jo = jo2 * 2
        load_wait(jo, 0)
        d0 = [pltpu.async_copy(buf.at[0, k], table.at[idv.at[0, k]], ss0,
                               add=True) for k in range(vb)]
        load_wait(jo + 1, 1)
        d1 = [pltpu.async_copy(buf.at[1, k], table.at[idv.at[1, k]], ss1,
                               add=True) for k in range(vb)]
        for d in d0:
            d.wait()

        @pl.when(jo + 2 < njo)
        def _():
            load(jo + 2, 0)

        for d in d1:
            d.wait()

        @pl.when(jo + 3 < njo)
        def _():
            load(jo + 3, 1)

        return 0

    lax.fori_loop(0, njo // 2, body, 0)
    plsc.subcore_barrier()
    pltpu.sync_copy(table.at[pl.ds(s * rs, rs)], out.at[c, s])


def _make_scatter(nseg, wd, vb):
    rs = nseg // NS
    return functools.partial(
        pl.kernel,
        out_type=jax.ShapeDtypeStruct((NC, NS, rs, wd), F32),
        mesh=_mesh,
        compiler_params=_sc_params,
        scratch_types=[
            pltpu.VMEM((2, vb, CHUNK), jnp.int32),
            pltpu.VMEM((2, vb, CHUNK, wd), F32),
            pltpu.VMEM_SHARED((nseg, wd), F32),
            pltpu.SemaphoreType.DMA,
            pltpu.SemaphoreType.DMA,
            pltpu.SemaphoreType.DMA,
            pltpu.SemaphoreType.DMA,
        ],
    )(functools.partial(_scatter_body, nseg, wd, vb))


VB_P = 7
VB_S = 14
_scatter_chk = _make_scatter(S_CHK, WP, VB_P)
_scatter_var = _make_scatter(S_VAR, B, VB_S)


def _gather_body(wd, vb, table, ids, out, idv, buf, sg0, sg1, so0, so1):
    njo = CPT // vb
    w = _wid()
    gsems = (sg0, sg1)
    osems = (so0, so1)

    def load_ids(jo, par):
        pltpu.async_copy(ids.at[w, pl.ds(jo * vb, vb)], idv.at[par], gsems[par])

    def ids_wait(jo, par):
        pltpu.make_async_copy(
            ids.at[w, pl.ds(jo * vb, vb)], idv.at[par], gsems[par]).wait()

    load_ids(0, 0)
    load_ids(1, 1)

    def body(jo2, _):
        jo = jo2 * 2

        @pl.when(jo2 >= 1)
        def _():
            pltpu.make_async_copy(buf.at[0], out.at[w, jo - 2], osems[0]).wait()

        ids_wait(jo, 0)
        d0 = [pltpu.async_copy(table.at[idv.at[0, k]], buf.at[0, k], gsems[0])
              for k in range(vb)]

        @pl.when(jo2 >= 1)
        def _():
            pltpu.make_async_copy(buf.at[1], out.at[w, jo - 1], osems[1]).wait()

        ids_wait(jo + 1, 1)
        d1 = [pltpu.async_copy(table.at[idv.at[1, k]], buf.at[1, k], gsems[1])
              for k in range(vb)]
        for d in d0:
            d.wait()
        pltpu.async_copy(buf.at[0], out.at[w, jo], osems[0])

        @pl.when(jo + 2 < njo)
        def _():
            load_ids(jo + 2, 0)

        for d in d1:
            d.wait()
        pltpu.async_copy(buf.at[1], out.at[w, jo + 1], osems[1])

        @pl.when(jo + 3 < njo)
        def _():
            load_ids(jo + 3, 1)

        return 0

    lax.fori_loop(0, njo // 2, body, 0)
    pltpu.make_async_copy(buf.at[0], out.at[w, njo - 2], osems[0]).wait()
    pltpu.make_async_copy(buf.at[1], out.at[w, njo - 1], osems[1]).wait()


def _make_gather(wd, vb):
    return functools.partial(
        pl.kernel,
        out_type=jax.ShapeDtypeStruct((NW, CPT // vb, vb, CHUNK, wd), F32),
        mesh=_mesh,
        compiler_params=_sc_params,
        scratch_types=[
            pltpu.VMEM((2, vb, CHUNK), jnp.int32),
            pltpu.VMEM((2, vb, CHUNK, wd), F32),
            pltpu.SemaphoreType.DMA,
            pltpu.SemaphoreType.DMA,
            pltpu.SemaphoreType.DMA,
            pltpu.SemaphoreType.DMA,
        ],
    )(functools.partial(_gather_body, wd, vb))


_gather_chk = _make_gather(WP, VB_P)
_gather_var = _make_gather(B, VB_S)


_RB = 448
_EW_GRID = EROWS // _RB
_ew_spec = pl.BlockSpec((_RB, 1024), lambda i: (i, 0))
_ew2_spec = pl.BlockSpec((_RB, 2048), lambda i: (i, 0))


def _lognegs(v2c):
    t = jnp.tanh(v2c * 0.5)
    mag = jnp.clip(jnp.abs(t), 1e-7, 0.999999)
    lm = jnp.log(mag)
    ng = jnp.where(t < 0.0, 1.0, 0.0).astype(F32)
    return lm, ng


def _e1_body(v2c_ref, out_ref):
    lm, ng = _lognegs(v2c_ref[...])
    lm3 = lm.reshape(_RB, 64, 16)
    ng3 = ng.reshape(_RB, 64, 16)
    out_ref[...] = jnp.concatenate([lm3, ng3], axis=2).reshape(_RB, 2048)


def _e1(v2c):
    return pl.pallas_call(
        _e1_body,
        grid=(_EW_GRID,),
        in_specs=[_ew_spec],
        out_specs=_ew2_spec,
        out_shape=jax.ShapeDtypeStruct((EROWS, 2048), F32),
    )(v2c)


def _e2_body(alpha_ref, v2c_ref, g_ref, c2v_ref):
    lm, ng = _lognegs(v2c_ref[...])
    g3 = g_ref[...].reshape(_RB, 64, 32)
    gl = g3[:, :, :16].reshape(_RB, 1024)
    gn = g3[:, :, 16:].reshape(_RB, 1024)
    loo_log = gl - lm
    loo_neg = gn - ng
    sign = 1.0 - 2.0 * jnp.mod(loo_neg, 2.0)
    prod = jnp.clip(sign * jnp.exp(loo_log), -0.999, 0.999)
    c2v_ref[...] = alpha_ref[0, 0] * jnp.log((1.0 + prod) / (1.0 - prod))


def _e2(alpha, v2c, g):
    return pl.pallas_call(
        _e2_body,
        grid=(_EW_GRID,),
        in_specs=[
            pl.BlockSpec((1, 1), lambda i: (0, 0), memory_space=pltpu.SMEM),
            _ew_spec, _ew2_spec,
        ],
        out_specs=_ew_spec,
        out_shape=jax.ShapeDtypeStruct((EROWS, 1024), F32),
    )(alpha.reshape(1, 1), v2c, g)


def _e3_body(ch_ref, g_ref, c2v_ref, out_ref):
    out_ref[...] = ch_ref[...] + g_ref[...] - c2v_ref[...]


def _e3(ch, g, c2v):
    return pl.pallas_call(
        _e3_body,
        grid=(_EW_GRID,),
        in_specs=[_ew_spec] * 3,
        out_specs=_ew_spec,
        out_shape=jax.ShapeDtypeStruct((EROWS, 1024), F32),
    )(ch, g, c2v)


def _combine_body(p_ref, out_ref):
    out_ref[...] = p_ref[0] + p_ref[1]


def _combine(partials, nseg, wd):
    rows = nseg * wd // 1024
    p = partials.reshape(2, rows, 1024)
    return pl.pallas_call(
        _combine_body,
        out_shape=jax.ShapeDtypeStruct((rows, 1024), F32),
    )(p)


def _final_body(llr_ref, tab_ref, out_ref):
    out_ref[...] = llr_ref[...] + tab_ref[...]


def _final(llr_flat, tab_flat):
    rows = S_VAR * B // 1024
    return pl.pallas_call(
        _final_body,
        out_shape=jax.ShapeDtypeStruct((rows, 1024), F32),
    )(llr_flat, tab_flat)


def kernel(channel_llrs, edge_index, alpha):
    ids32 = edge_index.astype(jnp.int32)
    pad = EPAD - NE
    var_ids = jnp.concatenate(
        [ids32[0], jnp.full((pad,), NV, jnp.int32)]).reshape(NW, CPT, CHUNK)
    chk_ids = jnp.concatenate(
        [ids32[1], jnp.full((pad,), NCK, jnp.int32)]).reshape(NW, CPT, CHUNK)

    llr_tab = jnp.pad(channel_llrs.astype(F32).T, ((0, S_VAR - NV), (0, 0)))
    llr_flat = llr_tab.reshape(S_VAR * B // 1024, 1024)
    z_chk = jnp.zeros((NS, S_CHK // NS, WP), F32)
    z_var = jnp.zeros((NS, S_VAR // NS, B), F32)

    def rows_b(flat):
        return flat.reshape(NW, CPT // VB_S, VB_S, CHUNK, B)

    def rows_p(flat2):
        return flat2.reshape(NW, CPT // VB_P, VB_P, CHUNK, WP)

    ch_e = _gather_var(llr_tab, var_ids).reshape(EROWS, 1024)
    v2c = ch_e
    tab_var = None
    for _ in range(ITERS):
        packed = _e1(v2c)
        p_chk = _scatter_chk(rows_p(packed), chk_ids, z_chk)
        tab_chk = _combine(p_chk, S_CHK, WP).reshape(S_CHK, WP)
        g_chk = _gather_chk(tab_chk, chk_ids).reshape(EROWS, 2048)
        c2v = _e2(alpha.astype(F32), v2c, g_chk)
        p_c2v = _scatter_var(rows_b(c2v), var_ids, z_var)
        tab_var = _combine(p_c2v, S_VAR, B)
        g_c2v = _gather_var(tab_var.reshape(S_VAR, B), var_ids).reshape(EROWS, 1024)
        v2c = _e3(ch_e, g_c2v, c2v)

    final = _final(llr_flat, tab_var).reshape(S_VAR, B)
    return final[:NV].T


# speedup vs baseline: 2.6892x; 2.6892x over previous
import functools

import jax
import jax.numpy as jnp
from jax import lax
from jax.experimental import pallas as pl
from jax.experimental.pallas import tpu as pltpu
from jax.experimental.pallas import tpu_sc as plsc

F32 = jnp.float32

NV = 50000
NCK = 25000
NE = 800000
B = 16
ITERS = 5

NC = 2
NS = 16
NW = NC * NS
CHUNK = 128
CPT = 196
EPAD = NW * CPT * CHUNK
VB = 14
VB_C = 7

S_CHK = 25024
S_VAR = 50048

_mesh = plsc.VectorSubcoreMesh(
    core_axis_name="c", subcore_axis_name="s", num_cores=NC, num_subcores=NS)
_sc_params = pltpu.CompilerParams(use_tc_tiling_on_sc=False)


def _wid():
    return lax.axis_index("s") * NC + lax.axis_index("c")


def _scatter_body(nseg, nv, vb, vals, ids, zeros, outs, idv, bufs, tables,
                  sl0, sl1, ss0, ss1):
    rs = nseg // NS
    njo = CPT // vb
    c = lax.axis_index("c")
    s = lax.axis_index("s")
    w = _wid()
    for t in tables:
        pltpu.sync_copy(zeros.at[s], t.at[pl.ds(s * rs, rs)])
    plsc.subcore_barrier()

    lsems = (sl0, sl1)
    ssems = (ss0, ss1)

    def load(jo, par):
        pltpu.async_copy(ids.at[w, pl.ds(jo * vb, vb)], idv.at[par],
                         lsems[par])
        for v, buf in zip(vals, bufs):
            pltpu.async_copy(v.at[w, jo], buf.at[par], lsems[par])

    def load_wait(jo, par):
        pltpu.make_async_copy(ids.at[w, pl.ds(jo * vb, vb)], idv.at[par],
                              lsems[par]).wait()
        for v, buf in zip(vals, bufs):
            pltpu.make_async_copy(v.at[w, jo], buf.at[par], lsems[par]).wait()

    def fire(par):
        return [pltpu.async_copy(buf.at[par, k], t.at[idv.at[par, k]],
                                 ssems[par], add=True)
                for buf, t in zip(bufs, tables) for k in range(vb)]

    load(0, 0)
    load(1, 1)

    def body(jo2, _):
        jo = jo2 * 2
        load_wait(jo, 0)
        d0 = fire(0)
        load_wait(jo + 1, 1)
        d1 = fire(1)
        for d in d0:
            d.wait()

        @pl.when(jo + 2 < njo)
        def _():
            load(jo + 2, 0)

        for d in d1:
            d.wait()

        @pl.when(jo + 3 < njo)
        def _():
            load(jo + 3, 1)

        return 0

    lax.fori_loop(0, njo // 2, body, 0)
    plsc.subcore_barrier()
    for t, o in zip(tables, outs):
        pltpu.sync_copy(t.at[pl.ds(s * rs, rs)], o.at[c, s])


def _scatter_wrap(nseg, nv, vb, body):
    rs = nseg // NS

    def wrapped(*args):
        vals = args[:nv]
        ids, zeros = args[nv], args[nv + 1]
        outs = args[nv + 2:nv + 2 + nv]
        idv = args[nv * 2 + 2]
        bufs = args[nv * 2 + 3:nv * 3 + 3]
        tables = args[nv * 3 + 3:nv * 4 + 3]
        sems = args[nv * 4 + 3:]
        body(nseg, nv, vb, vals, ids, zeros, outs, idv, bufs, tables, *sems)

    return functools.partial(
        pl.kernel,
        out_type=[jax.ShapeDtypeStruct((NC, NS, rs, B), F32)] * nv,
        mesh=_mesh,
        compiler_params=_sc_params,
        scratch_types=(
            [pltpu.VMEM((2, vb, CHUNK), jnp.int32)]
            + [pltpu.VMEM((2, vb, CHUNK, B), F32)] * nv
            + [pltpu.VMEM_SHARED((nseg, B), F32)] * nv
            + [pltpu.SemaphoreType.DMA] * 4
        ),
    )(wrapped)


_scatter_chk = _scatter_wrap(S_CHK, 2, VB_C, _scatter_body)
_scatter_var_raw = _scatter_wrap(S_VAR, 1, VB, _scatter_body)


def _scatter_var(vals, ids, zeros):
    (out,) = _scatter_var_raw(vals, ids, zeros)
    return out


def _gather_body(nv, tables, ids, outs, idv, bufs, sg0, sg1, so0, so1):
    njo = CPT // VB
    w = _wid()
    gsems = (sg0, sg1)
    osems = (so0, so1)

    def load_ids(jo, par):
        pltpu.async_copy(ids.at[w, pl.ds(jo * VB, VB)], idv.at[par],
                         gsems[par])

    def ids_wait(jo, par):
        pltpu.make_async_copy(ids.at[w, pl.ds(jo * VB, VB)], idv.at[par],
                              gsems[par]).wait()

    def fire(par):
        return [pltpu.async_copy(t.at[idv.at[par, k]], buf.at[par, k],
                                 gsems[par])
                for t, buf in zip(tables, bufs) for k in range(VB)]

    def out_wait(jo, par):
        for buf, o in zip(bufs, outs):
            pltpu.make_async_copy(buf.at[par], o.at[w, jo], osems[par]).wait()

    def out_fire(jo, par):
        for buf, o in zip(bufs, outs):
            pltpu.async_copy(buf.at[par], o.at[w, jo], osems[par])

    load_ids(0, 0)
    load_ids(1, 1)

    def body(jo2, _):
        jo = jo2 * 2

        @pl.when(jo2 >= 1)
        def _():
            out_wait(jo - 2, 0)

        ids_wait(jo, 0)
        d0 = fire(0)

        @pl.when(jo2 >= 1)
        def _():
            out_wait(jo - 1, 1)

        ids_wait(jo + 1, 1)
        d1 = fire(1)
        for d in d0:
            d.wait()
        out_fire(jo, 0)

        @pl.when(jo + 2 < njo)
        def _():
            load_ids(jo + 2, 0)

        for d in d1:
            d.wait()
        out_fire(jo + 1, 1)

        @pl.when(jo + 3 < njo)
        def _():
            load_ids(jo + 3, 1)

        return 0

    lax.fori_loop(0, njo // 2, body, 0)
    out_wait(njo - 2, 0)
    out_wait(njo - 1, 1)


def _gather_wrap(nv, body):
    def wrapped(*args):
        tables = args[:nv]
        ids = args[nv]
        outs = args[nv + 1:nv * 2 + 1]
        idv = args[nv * 2 + 1]
        bufs = args[nv * 2 + 2:nv * 3 + 2]
        sems = args[nv * 3 + 2:]
        body(nv, tables, ids, outs, idv, bufs, *sems)

    return functools.partial(
        pl.kernel,
        out_type=[jax.ShapeDtypeStruct((NW, CPT // VB, VB, CHUNK, B), F32)]
        * nv,
        mesh=_mesh,
        compiler_params=_sc_params,
        scratch_types=(
            [pltpu.VMEM((2, VB, CHUNK), jnp.int32)]
            + [pltpu.VMEM((2, VB, CHUNK, B), F32)] * nv
            + [pltpu.SemaphoreType.DMA] * 4
        ),
    )(wrapped)


_gather_chk = _gather_wrap(2, _gather_body)
_gather_var_raw = _gather_wrap(1, _gather_body)


def _gather_var(table, ids):
    (out,) = _gather_var_raw(table, ids)
    return out


XR1 = EPAD * B // 128
_RBX = 3584
_EW_GRID = XR1 // _RBX
_x1_spec = pl.BlockSpec((_RBX, 128), lambda i: (i, 0))


def _lognegs(v2c):
    t = jnp.tanh(v2c * 0.5)
    mag = jnp.clip(jnp.abs(t), 1e-7, 0.999999)
    lm = jnp.log(mag)
    ng = jnp.where(t < 0.0, 1.0, 0.0).astype(F32)
    return lm, ng


def _e1_body(v2c_ref, lm_ref, ng_ref):
    lm, ng = _lognegs(v2c_ref[...])
    lm_ref[...] = lm
    ng_ref[...] = ng


def _e1(v2c):
    return pl.pallas_call(
        _e1_body,
        grid=(_EW_GRID,),
        in_specs=[_x1_spec],
        out_specs=[_x1_spec, _x1_spec],
        out_shape=[jax.ShapeDtypeStruct((XR1, 128), F32)] * 2,
    )(v2c)


def _e2_body(alpha_ref, v2c_ref, gl_ref, gn_ref, c2v_ref):
    lm, ng = _lognegs(v2c_ref[...])
    loo_log = gl_ref[...] - lm
    loo_neg = gn_ref[...] - ng
    sign = 1.0 - 2.0 * jnp.mod(loo_neg, 2.0)
    prod = jnp.clip(sign * jnp.exp(loo_log), -0.999, 0.999)
    c2v_ref[...] = alpha_ref[0, 0] * jnp.log((1.0 + prod) / (1.0 - prod))


def _e2(alpha, v2c, gl, gn):
    return pl.pallas_call(
        _e2_body,
        grid=(_EW_GRID,),
        in_specs=[
            pl.BlockSpec((1, 1), lambda i: (0, 0), memory_space=pltpu.SMEM),
            _x1_spec, _x1_spec, _x1_spec,
        ],
        out_specs=_x1_spec,
        out_shape=jax.ShapeDtypeStruct((XR1, 128), F32),
    )(alpha.reshape(1, 1), v2c, gl, gn)


def _e3_body(ch_ref, g_ref, c2v_ref, out_ref):
    out_ref[...] = ch_ref[...] + g_ref[...] - c2v_ref[...]


def _e3(ch, g, c2v):
    return pl.pallas_call(
        _e3_body,
        grid=(_EW_GRID,),
        in_specs=[_x1_spec] * 3,
        out_specs=_x1_spec,
        out_shape=jax.ShapeDtypeStruct((XR1, 128), F32),
    )(ch, g, c2v)


def _combine_body(p_ref, out_ref):
    out_ref[...] = p_ref[0] + p_ref[1]


def _combine(partials, nseg):
    rows = nseg * B // 128
    p = partials.reshape(2, rows, 128)
    return pl.pallas_call(
        _combine_body,
        out_shape=jax.ShapeDtypeStruct((rows, 128), F32),
    )(p)


def _final_body(llr_ref, tab_ref, out_ref):
    out_ref[...] = llr_ref[...] + tab_ref[...]


def _final(llr_flat, tab_flat):
    rows = S_VAR * B // 128
    return pl.pallas_call(
        _final_body,
        out_shape=jax.ShapeDtypeStruct((rows, 128), F32),
    )(llr_flat, tab_flat)


def kernel(channel_llrs, edge_index, alpha):
    ids32 = edge_index.astype(jnp.int32)
    pad = EPAD - NE
    var_ids = jnp.concatenate(
        [ids32[0], jnp.full((pad,), NV, jnp.int32)]).reshape(NW, CPT, CHUNK)
    chk_ids = jnp.concatenate(
        [ids32[1], jnp.full((pad,), NCK, jnp.int32)]).reshape(NW, CPT, CHUNK)

    llr_tab = jnp.pad(channel_llrs.astype(F32).T, ((0, S_VAR - NV), (0, 0)))
    llr_flat = llr_tab.reshape(S_VAR * B // 128, 128)
    z_chk = jnp.zeros((NS, S_CHK // NS, B), F32)
    z_var = jnp.zeros((NS, S_VAR // NS, B), F32)

    def rows5(flat, vb=VB):
        return flat.reshape(NW, CPT // vb, vb, CHUNK, B)

    def flat2(x):
        return x.reshape(XR1, 128)

    ch_e = flat2(_gather_var(llr_tab, var_ids))
    v2c = ch_e
    tab_var = None
    for _ in range(ITERS):
        lm, ng = _e1(v2c)
        p_log, p_neg = _scatter_chk(rows5(lm, VB_C), rows5(ng, VB_C), chk_ids, z_chk)
        tab_log = _combine(p_log, S_CHK).reshape(S_CHK, B)
        tab_neg = _combine(p_neg, S_CHK).reshape(S_CHK, B)
        g_log, g_neg = _gather_chk(tab_log, tab_neg, chk_ids)
        c2v = _e2(alpha.astype(F32), v2c, flat2(g_log), flat2(g_neg))
        p_c2v = _scatter_var(rows5(c2v), var_ids, z_var)
        tab_var = _combine(p_c2v, S_VAR)
        g_c2v = flat2(_gather_var(tab_var.reshape(S_VAR, B), var_ids))
        v2c = _e3(ch_e, g_c2v, c2v)

    final = _final(llr_flat, tab_var).reshape(S_VAR, B)
    return final[:NV].T


# speedup vs baseline: 2.7759x; 1.0322x over previous
import functools

import jax
import jax.numpy as jnp
from jax import lax
from jax.experimental import pallas as pl
from jax.experimental.pallas import tpu as pltpu
from jax.experimental.pallas import tpu_sc as plsc

F32 = jnp.float32

NV = 50000
NCK = 25000
NE = 800000
B = 16
ITERS = 5

NC = 2
NS = 16
NW = NC * NS
CHUNK = 128
CPT = 196
EPAD = NW * CPT * CHUNK
VB = 14
VB_C = 7

S_CHK = 25024
S_VAR = 50048

_mesh = plsc.VectorSubcoreMesh(
    core_axis_name="c", subcore_axis_name="s", num_cores=NC, num_subcores=NS)
_sc_params = pltpu.CompilerParams(use_tc_tiling_on_sc=False)


def _wid():
    return lax.axis_index("s") * NC + lax.axis_index("c")


def _scatter_body(nseg, nv, vb, vals, ids, zeros, outs, idv, bufs, tables,
                  sl0, sl1, ss0, ss1):
    rs = nseg // NS
    njo = CPT // vb
    c = lax.axis_index("c")
    s = lax.axis_index("s")
    w = _wid()
    for t in tables:
        pltpu.sync_copy(zeros.at[s], t.at[pl.ds(s * rs, rs)])
    plsc.subcore_barrier()

    lsems = (sl0, sl1)
    ssems = (ss0, ss1)

    def load(jo, par):
        pltpu.async_copy(ids.at[w, pl.ds(jo * vb, vb)], idv.at[par],
                         lsems[par])
        for v, buf in zip(vals, bufs):
            pltpu.async_copy(v.at[w, jo], buf.at[par], lsems[par])

    def load_wait(jo, par):
        pltpu.make_async_copy(ids.at[w, pl.ds(jo * vb, vb)], idv.at[par],
                              lsems[par]).wait()
        for v, buf in zip(vals, bufs):
            pltpu.make_async_copy(v.at[w, jo], buf.at[par], lsems[par]).wait()

    def fire(par):
        return [pltpu.async_copy(buf.at[par, k], t.at[idv.at[par, k]],
                                 ssems[par], add=True)
                for buf, t in zip(bufs, tables) for k in range(vb)]

    load(0, 0)
    load(1, 1)

    def body(jo2, _):
        jo = jo2 * 2
        load_wait(jo, 0)
        d0 = fire(0)
        load_wait(jo + 1, 1)
        d1 = fire(1)
        for d in d0:
            d.wait()

        @pl.when(jo + 2 < njo)
        def _():
            load(jo + 2, 0)

        for d in d1:
            d.wait()

        @pl.when(jo + 3 < njo)
        def _():
            load(jo + 3, 1)

        return 0

    lax.fori_loop(0, njo // 2, body, 0)
    plsc.subcore_barrier()
    for t, o in zip(tables, outs):
        pltpu.sync_copy(t.at[pl.ds(s * rs, rs)], o.at[c, s])


def _scatter_wrap(nseg, nv, vb, body):
    rs = nseg // NS

    def wrapped(*args):
        vals = args[:nv]
        ids, zeros = args[nv], args[nv + 1]
        outs = args[nv + 2:nv + 2 + nv]
        idv = args[nv * 2 + 2]
        bufs = args[nv * 2 + 3:nv * 3 + 3]
        tables = args[nv * 3 + 3:nv * 4 + 3]
        sems = args[nv * 4 + 3:]
        body(nseg, nv, vb, vals, ids, zeros, outs, idv, bufs, tables, *sems)

    return functools.partial(
        pl.kernel,
        out_type=[jax.ShapeDtypeStruct((NC, NS, rs, B), F32)] * nv,
        mesh=_mesh,
        compiler_params=_sc_params,
        scratch_types=(
            [pltpu.VMEM((2, vb, CHUNK), jnp.int32)]
            + [pltpu.VMEM((2, vb, CHUNK, B), F32)] * nv
            + [pltpu.VMEM_SHARED((nseg, B), F32)] * nv
            + [pltpu.SemaphoreType.DMA] * 4
        ),
    )(wrapped)


_scatter_chk = _scatter_wrap(S_CHK, 2, VB_C, _scatter_body)
_scatter_var_raw = _scatter_wrap(S_VAR, 1, VB, _scatter_body)


def _scatter_var(vals, ids, zeros):
    (out,) = _scatter_var_raw(vals, ids, zeros)
    return out


def _gather_body(nv, tables, ids, outs, idv, bufs, sg0, sg1, so0, so1):
    njo = CPT // VB
    w = _wid()
    gsems = (sg0, sg1)
    osems = (so0, so1)

    def load_ids(jo, par):
        pltpu.async_copy(ids.at[w, pl.ds(jo * VB, VB)], idv.at[par],
                         gsems[par])

    def ids_wait(jo, par):
        pltpu.make_async_copy(ids.at[w, pl.ds(jo * VB, VB)], idv.at[par],
                              gsems[par]).wait()

    def fire(par):
        return [pltpu.async_copy(t.at[idv.at[par, k]], buf.at[par, k],
                                 gsems[par])
                for t, buf in zip(tables, bufs) for k in range(VB)]

    def out_wait(jo, par):
        for buf, o in zip(bufs, outs):
            pltpu.make_async_copy(buf.at[par], o.at[w, jo], osems[par]).wait()

    def out_fire(jo, par):
        for buf, o in zip(bufs, outs):
            pltpu.async_copy(buf.at[par], o.at[w, jo], osems[par])

    load_ids(0, 0)
    load_ids(1, 1)

    def body(jo2, _):
        jo = jo2 * 2

        @pl.when(jo2 >= 1)
        def _():
            out_wait(jo - 2, 0)

        ids_wait(jo, 0)
        d0 = fire(0)

        @pl.when(jo2 >= 1)
        def _():
            out_wait(jo - 1, 1)

        ids_wait(jo + 1, 1)
        d1 = fire(1)
        for d in d0:
            d.wait()
        out_fire(jo, 0)

        @pl.when(jo + 2 < njo)
        def _():
            load_ids(jo + 2, 0)

        for d in d1:
            d.wait()
        out_fire(jo + 1, 1)

        @pl.when(jo + 3 < njo)
        def _():
            load_ids(jo + 3, 1)

        return 0

    lax.fori_loop(0, njo // 2, body, 0)
    out_wait(njo - 2, 0)
    out_wait(njo - 1, 1)


def _gather_wrap(nv, body):
    def wrapped(*args):
        tables = args[:nv]
        ids = args[nv]
        outs = args[nv + 1:nv * 2 + 1]
        idv = args[nv * 2 + 1]
        bufs = args[nv * 2 + 2:nv * 3 + 2]
        sems = args[nv * 3 + 2:]
        body(nv, tables, ids, outs, idv, bufs, *sems)

    return functools.partial(
        pl.kernel,
        out_type=[jax.ShapeDtypeStruct((NW, CPT // VB, VB, CHUNK, B), F32)]
        * nv,
        mesh=_mesh,
        compiler_params=_sc_params,
        scratch_types=(
            [pltpu.VMEM((2, VB, CHUNK), jnp.int32)]
            + [pltpu.VMEM((2, VB, CHUNK, B), F32)] * nv
            + [pltpu.SemaphoreType.DMA] * 4
        ),
    )(wrapped)


_gather_chk = _gather_wrap(2, _gather_body)
_gather_var_raw = _gather_wrap(1, _gather_body)


def _gather_var(table, ids):
    (out,) = _gather_var_raw(table, ids)
    return out


XR1 = EPAD * B // 128
_RBX = 3584
_EW_GRID = XR1 // _RBX
_x1_spec = pl.BlockSpec((_RBX, 128), lambda i: (i, 0))


def _lognegs(v2c):
    t = jnp.tanh(v2c * 0.5)
    mag = jnp.clip(jnp.abs(t), 1e-7, 0.999999)
    lm = jnp.log(mag)
    ng = jnp.where(t < 0.0, 1.0, 0.0).astype(F32)
    return lm, ng


def _e1_body(v2c_ref, lm_ref, ng_ref):
    lm, ng = _lognegs(v2c_ref[...])
    lm_ref[...] = lm
    ng_ref[...] = ng


def _e1(v2c):
    return pl.pallas_call(
        _e1_body,
        grid=(_EW_GRID,),
        in_specs=[_x1_spec],
        out_specs=[_x1_spec, _x1_spec],
        out_shape=[jax.ShapeDtypeStruct((XR1, 128), F32)] * 2,
    )(v2c)


def _e2_body(alpha_ref, v2c_ref, gl_ref, gn_ref, c2v_ref):
    lm, ng = _lognegs(v2c_ref[...])
    loo_log = gl_ref[...] - lm
    loo_neg = gn_ref[...] - ng
    sign = 1.0 - 2.0 * jnp.mod(loo_neg, 2.0)
    prod = jnp.clip(sign * jnp.exp(loo_log), -0.999, 0.999)
    c2v_ref[...] = alpha_ref[0, 0] * jnp.log((1.0 + prod) / (1.0 - prod))


def _e2(alpha, v2c, gl, gn):
    return pl.pallas_call(
        _e2_body,
        grid=(_EW_GRID,),
        in_specs=[
            pl.BlockSpec((1, 1), lambda i: (0, 0), memory_space=pltpu.SMEM),
            _x1_spec, _x1_spec, _x1_spec,
        ],
        out_specs=_x1_spec,
        out_shape=jax.ShapeDtypeStruct((XR1, 128), F32),
    )(alpha.reshape(1, 1), v2c, gl, gn)


def _e13_body(ch_ref, g_ref, c2v_ref, v2c_ref, lm_ref, ng_ref):
    v2c = ch_ref[...] + g_ref[...] - c2v_ref[...]
    v2c_ref[...] = v2c
    lm, ng = _lognegs(v2c)
    lm_ref[...] = lm
    ng_ref[...] = ng


def _e13(ch, g, c2v):
    return pl.pallas_call(
        _e13_body,
        grid=(_EW_GRID,),
        in_specs=[_x1_spec] * 3,
        out_specs=[_x1_spec] * 3,
        out_shape=[jax.ShapeDtypeStruct((XR1, 128), F32)] * 3,
    )(ch, g, c2v)


def _combine2_body(pa_ref, pb_ref, oa_ref, ob_ref):
    oa_ref[...] = pa_ref[0] + pa_ref[1]
    ob_ref[...] = pb_ref[0] + pb_ref[1]


def _combine2(pa, pb, nseg):
    rows = nseg * B // 128
    return pl.pallas_call(
        _combine2_body,
        out_shape=[jax.ShapeDtypeStruct((rows, 128), F32)] * 2,
    )(pa.reshape(2, rows, 128), pb.reshape(2, rows, 128))


def _combine_body(p_ref, out_ref):
    out_ref[...] = p_ref[0] + p_ref[1]


def _combine(partials, nseg):
    rows = nseg * B // 128
    p = partials.reshape(2, rows, 128)
    return pl.pallas_call(
        _combine_body,
        out_shape=jax.ShapeDtypeStruct((rows, 128), F32),
    )(p)


def _final_body(llr_ref, p_ref, out_ref):
    out_ref[...] = llr_ref[...] + p_ref[0] + p_ref[1]


def _final(llr_flat, p_var):
    rows = S_VAR * B // 128
    return pl.pallas_call(
        _final_body,
        out_shape=jax.ShapeDtypeStruct((rows, 128), F32),
    )(llr_flat, p_var.reshape(2, rows, 128))


def kernel(channel_llrs, edge_index, alpha):
    ids32 = edge_index.astype(jnp.int32)
    pad = EPAD - NE
    var_ids = jnp.concatenate(
        [ids32[0], jnp.full((pad,), NV, jnp.int32)]).reshape(NW, CPT, CHUNK)
    chk_ids = jnp.concatenate(
        [ids32[1], jnp.full((pad,), NCK, jnp.int32)]).reshape(NW, CPT, CHUNK)

    llr_tab = jnp.pad(channel_llrs.astype(F32).T, ((0, S_VAR - NV), (0, 0)))
    llr_flat = llr_tab.reshape(S_VAR * B // 128, 128)
    z_chk = jnp.zeros((NS, S_CHK // NS, B), F32)
    z_var = jnp.zeros((NS, S_VAR // NS, B), F32)

    def rows5(flat, vb=VB):
        return flat.reshape(NW, CPT // vb, vb, CHUNK, B)

    def flat2(x):
        return x.reshape(XR1, 128)

    ch_e = flat2(_gather_var(llr_tab, var_ids))
    v2c = ch_e
    lm, ng = _e1(v2c)
    p_c2v = None
    for it in range(ITERS):
        p_log, p_neg = _scatter_chk(rows5(lm, VB_C), rows5(ng, VB_C),
                                    chk_ids, z_chk)
        tab_log, tab_neg = _combine2(p_log, p_neg, S_CHK)
        g_log, g_neg = _gather_chk(tab_log.reshape(S_CHK, B),
                                   tab_neg.reshape(S_CHK, B), chk_ids)
        c2v = _e2(alpha.astype(F32), v2c, flat2(g_log), flat2(g_neg))
        p_c2v = _scatter_var(rows5(c2v), var_ids, z_var)
        if it < ITERS - 1:
            tab_var = _combine(p_c2v, S_VAR)
            g_c2v = flat2(_gather_var(tab_var.reshape(S_VAR, B), var_ids))
            v2c, lm, ng = _e13(ch_e, g_c2v, c2v)

    final = _final(llr_flat, p_c2v).reshape(S_VAR, B)
    return final[:NV].T


# speedup vs baseline: 2.8944x; 1.0427x over previous
import functools

import jax
import jax.numpy as jnp
from jax import lax
from jax.experimental import pallas as pl
from jax.experimental.pallas import tpu as pltpu
from jax.experimental.pallas import tpu_sc as plsc

F32 = jnp.float32

NV = 50000
NCK = 25000
NE = 800000
B = 16
ITERS = 5

NC = 2
NS = 16
NW = NC * NS
CHUNK = 128
CPT = 196
EPAD = NW * CPT * CHUNK
VB = 14
VB_C = 7

S_CHK = 25024
S_VAR = 50048

_mesh = plsc.VectorSubcoreMesh(
    core_axis_name="c", subcore_axis_name="s", num_cores=NC, num_subcores=NS)
_sc_params = pltpu.CompilerParams(use_tc_tiling_on_sc=False)


def _wid():
    return lax.axis_index("s") * NC + lax.axis_index("c")


def _scatter_body(nseg, nv, vb, hs, vals, ids, zeros, outs, idv, bufs, tables,
                  sl0, sl1, ss0, ss1):
    rs = nseg // NS
    njo = CPT // vb
    c = lax.axis_index("c")
    s = lax.axis_index("s")
    w = _wid()
    for t in tables:
        pltpu.sync_copy(zeros.at[s], t.at[pl.ds(s * rs, rs)])
    plsc.subcore_barrier()

    lsems = (sl0, sl1)
    ssems = (ss0, ss1)

    def load(jo, par):
        pltpu.async_copy(ids.at[w, pl.ds(jo * vb, vb)], idv.at[par],
                         lsems[par])
        for v, buf in zip(vals, bufs):
            pltpu.async_copy(v.at[w, jo], buf.at[par], lsems[par])

    def load_wait(jo, par):
        pltpu.make_async_copy(ids.at[w, pl.ds(jo * vb, vb)], idv.at[par],
                              lsems[par]).wait()
        for v, buf in zip(vals, bufs):
            pltpu.make_async_copy(v.at[w, jo], buf.at[par], lsems[par]).wait()

    def fire(par):
        return [pltpu.async_copy(buf.at[par, k, h], t.at[idv.at[par, k, h]],
                                 ssems[par], add=True)
                for buf, t in zip(bufs, tables)
                for k in range(vb) for h in range(hs)]

    load(0, 0)
    load(1, 1)

    def body(jo2, _):
        jo = jo2 * 2
        load_wait(jo, 0)
        d0 = fire(0)
        load_wait(jo + 1, 1)
        d1 = fire(1)
        for d in d0:
            d.wait()

        @pl.when(jo + 2 < njo)
        def _():
            load(jo + 2, 0)

        for d in d1:
            d.wait()

        @pl.when(jo + 3 < njo)
        def _():
            load(jo + 3, 1)

        return 0

    lax.fori_loop(0, njo // 2, body, 0)
    plsc.subcore_barrier()
    for t, o in zip(tables, outs):
        pltpu.sync_copy(t.at[pl.ds(s * rs, rs)], o.at[c, s])


def _scatter_wrap(nseg, nv, vb, hs, body):
    rs = nseg // NS

    def wrapped(*args):
        vals = args[:nv]
        ids, zeros = args[nv], args[nv + 1]
        outs = args[nv + 2:nv + 2 + nv]
        idv = args[nv * 2 + 2]
        bufs = args[nv * 2 + 3:nv * 3 + 3]
        tables = args[nv * 3 + 3:nv * 4 + 3]
        sems = args[nv * 4 + 3:]
        body(nseg, nv, vb, hs, vals, ids, zeros, outs, idv, bufs, tables, *sems)

    return functools.partial(
        pl.kernel,
        out_type=[jax.ShapeDtypeStruct((NC, NS, rs, B), F32)] * nv,
        mesh=_mesh,
        compiler_params=_sc_params,
        scratch_types=(
            [pltpu.VMEM((2, vb, hs, CHUNK // hs), jnp.int32)]
            + [pltpu.VMEM((2, vb, hs, CHUNK // hs, B), F32)] * nv
            + [pltpu.VMEM_SHARED((nseg, B), F32)] * nv
            + [pltpu.SemaphoreType.DMA] * 4
        ),
    )(wrapped)


_scatter_chk = _scatter_wrap(S_CHK, 2, VB_C, 2, _scatter_body)
_scatter_var_raw = _scatter_wrap(S_VAR, 1, VB, 2, _scatter_body)


def _scatter_var(vals, ids, zeros):
    (out,) = _scatter_var_raw(vals, ids, zeros)
    return out


def _gather_body(nv, hs, tables, ids, outs, idv, bufs, sg0, sg1, so0, so1):
    njo = CPT // VB
    w = _wid()
    gsems = (sg0, sg1)
    osems = (so0, so1)

    def load_ids(jo, par):
        pltpu.async_copy(ids.at[w, pl.ds(jo * VB, VB)], idv.at[par],
                         gsems[par])

    def ids_wait(jo, par):
        pltpu.make_async_copy(ids.at[w, pl.ds(jo * VB, VB)], idv.at[par],
                              gsems[par]).wait()

    def fire(par):
        return [pltpu.async_copy(t.at[idv.at[par, k, h]], buf.at[par, k, h],
                                 gsems[par])
                for t, buf in zip(tables, bufs)
                for k in range(VB) for h in range(hs)]

    def out_wait(jo, par):
        for buf, o in zip(bufs, outs):
            pltpu.make_async_copy(buf.at[par], o.at[w, jo], osems[par]).wait()

    def out_fire(jo, par):
        for buf, o in zip(bufs, outs):
            pltpu.async_copy(buf.at[par], o.at[w, jo], osems[par])

    load_ids(0, 0)
    load_ids(1, 1)

    def body(jo2, _):
        jo = jo2 * 2

        @pl.when(jo2 >= 1)
        def _():
            out_wait(jo - 2, 0)

        ids_wait(jo, 0)
        d0 = fire(0)

        @pl.when(jo2 >= 1)
        def _():
            out_wait(jo - 1, 1)

        ids_wait(jo + 1, 1)
        d1 = fire(1)
        for d in d0:
            d.wait()
        out_fire(jo, 0)

        @pl.when(jo + 2 < njo)
        def _():
            load_ids(jo + 2, 0)

        for d in d1:
            d.wait()
        out_fire(jo + 1, 1)

        @pl.when(jo + 3 < njo)
        def _():
            load_ids(jo + 3, 1)

        return 0

    lax.fori_loop(0, njo // 2, body, 0)
    out_wait(njo - 2, 0)
    out_wait(njo - 1, 1)


def _gather_wrap(nv, hs, body):
    def wrapped(*args):
        tables = args[:nv]
        ids = args[nv]
        outs = args[nv + 1:nv * 2 + 1]
        idv = args[nv * 2 + 1]
        bufs = args[nv * 2 + 2:nv * 3 + 2]
        sems = args[nv * 3 + 2:]
        body(nv, hs, tables, ids, outs, idv, bufs, *sems)

    return functools.partial(
        pl.kernel,
        out_type=[jax.ShapeDtypeStruct((NW, CPT // VB, VB, hs, CHUNK // hs,
                                        B), F32)] * nv,
        mesh=_mesh,
        compiler_params=_sc_params,
        scratch_types=(
            [pltpu.VMEM((2, VB, hs, CHUNK // hs), jnp.int32)]
            + [pltpu.VMEM((2, VB, hs, CHUNK // hs, B), F32)] * nv
            + [pltpu.SemaphoreType.DMA] * 4
        ),
    )(wrapped)


_gather_chk = _gather_wrap(2, 2, _gather_body)
_gather_var_raw = _gather_wrap(1, 2, _gather_body)


def _gather_var(table, ids):
    (out,) = _gather_var_raw(table, ids)
    return out


XR1 = EPAD * B // 128
_RBX = 3584
_EW_GRID = XR1 // _RBX
_x1_spec = pl.BlockSpec((_RBX, 128), lambda i: (i, 0))


def _lognegs(v2c):
    t = jnp.tanh(v2c * 0.5)
    mag = jnp.clip(jnp.abs(t), 1e-7, 0.999999)
    lm = jnp.log(mag)
    ng = jnp.where(t < 0.0, 1.0, 0.0).astype(F32)
    return lm, ng


def _e1_body(v2c_ref, lm_ref, ng_ref):
    lm, ng = _lognegs(v2c_ref[...])
    lm_ref[...] = lm
    ng_ref[...] = ng


def _e1(v2c):
    return pl.pallas_call(
        _e1_body,
        grid=(_EW_GRID,),
        in_specs=[_x1_spec],
        out_specs=[_x1_spec, _x1_spec],
        out_shape=[jax.ShapeDtypeStruct((XR1, 128), F32)] * 2,
    )(v2c)


def _e2_body(alpha_ref, lm_ref, ng_ref, gl_ref, gn_ref, c2v_ref):
    loo_log = gl_ref[...] - lm_ref[...]
    loo_neg = gn_ref[...] - ng_ref[...]
    sign = 1.0 - 2.0 * jnp.mod(loo_neg, 2.0)
    prod = jnp.clip(sign * jnp.exp(loo_log), -0.999, 0.999)
    c2v_ref[...] = alpha_ref[0, 0] * jnp.log((1.0 + prod) / (1.0 - prod))


def _e2(alpha, lm, ng, gl, gn):
    return pl.pallas_call(
        _e2_body,
        grid=(_EW_GRID,),
        in_specs=[
            pl.BlockSpec((1, 1), lambda i: (0, 0), memory_space=pltpu.SMEM),
            _x1_spec, _x1_spec, _x1_spec, _x1_spec,
        ],
        out_specs=_x1_spec,
        out_shape=jax.ShapeDtypeStruct((XR1, 128), F32),
    )(alpha.reshape(1, 1), lm, ng, gl, gn)


def _e13_body(ch_ref, g_ref, c2v_ref, lm_ref, ng_ref):
    v2c = ch_ref[...] + g_ref[...] - c2v_ref[...]
    lm, ng = _lognegs(v2c)
    lm_ref[...] = lm
    ng_ref[...] = ng


def _e13(ch, g, c2v):
    return pl.pallas_call(
        _e13_body,
        grid=(_EW_GRID,),
        in_specs=[_x1_spec] * 3,
        out_specs=[_x1_spec] * 2,
        out_shape=[jax.ShapeDtypeStruct((XR1, 128), F32)] * 2,
    )(ch, g, c2v)


def _combine2_body(pa_ref, pb_ref, oa_ref, ob_ref):
    oa_ref[...] = pa_ref[0] + pa_ref[1]
    ob_ref[...] = pb_ref[0] + pb_ref[1]


def _combine2(pa, pb, nseg):
    rows = nseg * B // 128
    return pl.pallas_call(
        _combine2_body,
        out_shape=[jax.ShapeDtypeStruct((rows, 128), F32)] * 2,
    )(pa.reshape(2, rows, 128), pb.reshape(2, rows, 128))


def _combine_body(p_ref, out_ref):
    out_ref[...] = p_ref[0] + p_ref[1]


def _combine(partials, nseg):
    rows = nseg * B // 128
    p = partials.reshape(2, rows, 128)
    return pl.pallas_call(
        _combine_body,
        out_shape=jax.ShapeDtypeStruct((rows, 128), F32),
    )(p)


def _final_body(llr_ref, p_ref, out_ref):
    out_ref[...] = llr_ref[...] + p_ref[0] + p_ref[1]


def _final(llr_flat, p_var):
    rows = S_VAR * B // 128
    return pl.pallas_call(
        _final_body,
        out_shape=jax.ShapeDtypeStruct((rows, 128), F32),
    )(llr_flat, p_var.reshape(2, rows, 128))


def kernel(channel_llrs, edge_index, alpha):
    ids32 = edge_index.astype(jnp.int32)
    pad = EPAD - NE
    var_ids = jnp.concatenate(
        [ids32[0], jnp.full((pad,), NV, jnp.int32)]).reshape(NW, CPT, 2,
                                                             CHUNK // 2)
    chk_ids = jnp.concatenate(
        [ids32[1], jnp.full((pad,), NCK, jnp.int32)]).reshape(NW, CPT, 2,
                                                              CHUNK // 2)

    llr_tab = jnp.pad(channel_llrs.astype(F32).T, ((0, S_VAR - NV), (0, 0)))
    llr_flat = llr_tab.reshape(S_VAR * B // 128, 128)
    z_chk = jnp.zeros((NS, S_CHK // NS, B), F32)
    z_var = jnp.zeros((NS, S_VAR // NS, B), F32)

    def rows5(flat, vb=VB):
        return flat.reshape(NW, CPT // vb, vb, 2, CHUNK // 2, B)

    def flat2(x):
        return x.reshape(XR1, 128)

    ch_e = flat2(_gather_var(llr_tab, var_ids))
    lm, ng = _e1(ch_e)
    p_c2v = None
    for it in range(ITERS):
        p_log, p_neg = _scatter_chk(rows5(lm, VB_C), rows5(ng, VB_C),
                                    chk_ids, z_chk)
        tab_log, tab_neg = _combine2(p_log, p_neg, S_CHK)
        g_log, g_neg = _gather_chk(tab_log.reshape(S_CHK, B),
                                   tab_neg.reshape(S_CHK, B), chk_ids)
        c2v = _e2(alpha.astype(F32), lm, ng, flat2(g_log), flat2(g_neg))
        p_c2v = _scatter_var(rows5(c2v), var_ids, z_var)
        if it < ITERS - 1:
            tab_var = _combine(p_c2v, S_VAR)
            g_c2v = flat2(_gather_var(tab_var.reshape(S_VAR, B), var_ids))
            lm, ng = _e13(ch_e, g_c2v, c2v)

    final = _final(llr_flat, p_c2v).reshape(S_VAR, B)
    return final[:NV].T


# speedup vs baseline: 2.9255x; 1.0107x over previous
import functools

import jax
import jax.numpy as jnp
from jax import lax
from jax.experimental import pallas as pl
from jax.experimental.pallas import tpu as pltpu
from jax.experimental.pallas import tpu_sc as plsc

F32 = jnp.float32

NV = 50000
NCK = 25000
NE = 800000
B = 16
ITERS = 5

NC = 2
NS = 16
NW = NC * NS
CHUNK = 128
CPT = 196
EPAD = NW * CPT * CHUNK
VB = 14
VB_C = 7

S_CHK = 25024
S_VAR = 50048

_mesh = plsc.VectorSubcoreMesh(
    core_axis_name="c", subcore_axis_name="s", num_cores=NC, num_subcores=NS)
_sc_params = pltpu.CompilerParams(use_tc_tiling_on_sc=False)


def _wid():
    return lax.axis_index("s") * NC + lax.axis_index("c")


def _scatter_body(nseg, nv, vb, hs, cpt, vals, ids, zeros, outs, idv, bufs,
                  tables, sl0, sl1, ss0, ss1):
    rs = nseg // NS
    njo = cpt // vb
    c = lax.axis_index("c")
    s = lax.axis_index("s")
    w = _wid()
    for t in tables:
        pltpu.sync_copy(zeros.at[s], t.at[pl.ds(s * rs, rs)])
    plsc.subcore_barrier()

    lsems = (sl0, sl1)
    ssems = (ss0, ss1)

    def load(jo, par):
        pltpu.async_copy(ids.at[w, pl.ds(jo * vb, vb)], idv.at[par],
                         lsems[par])
        for v, buf in zip(vals, bufs):
            pltpu.async_copy(v.at[w, jo], buf.at[par], lsems[par])

    def load_wait(jo, par):
        pltpu.make_async_copy(ids.at[w, pl.ds(jo * vb, vb)], idv.at[par],
                              lsems[par]).wait()
        for v, buf in zip(vals, bufs):
            pltpu.make_async_copy(v.at[w, jo], buf.at[par], lsems[par]).wait()

    def fire(par):
        return [pltpu.async_copy(buf.at[par, k, h], t.at[idv.at[par, k, h]],
                                 ssems[par], add=True)
                for buf, t in zip(bufs, tables)
                for k in range(vb) for h in range(hs)]

    load(0, 0)
    load(1, 1)

    def body(jo2, _):
        jo = jo2 * 2
        load_wait(jo, 0)
        d0 = fire(0)
        load_wait(jo + 1, 1)
        d1 = fire(1)
        for d in d0:
            d.wait()

        @pl.when(jo + 2 < njo)
        def _():
            load(jo + 2, 0)

        for d in d1:
            d.wait()

        @pl.when(jo + 3 < njo)
        def _():
            load(jo + 3, 1)

        return 0

    lax.fori_loop(0, njo // 2, body, 0)
    plsc.subcore_barrier()
    for t, o in zip(tables, outs):
        pltpu.sync_copy(t.at[pl.ds(s * rs, rs)], o.at[c, s])


def _scatter_wrap(nseg, nv, vb, hs, cpt, body):
    rs = nseg // NS

    def wrapped(*args):
        vals = args[:nv]
        ids, zeros = args[nv], args[nv + 1]
        outs = args[nv + 2:nv + 2 + nv]
        idv = args[nv * 2 + 2]
        bufs = args[nv * 2 + 3:nv * 3 + 3]
        tables = args[nv * 3 + 3:nv * 4 + 3]
        sems = args[nv * 4 + 3:]
        body(nseg, nv, vb, hs, cpt, vals, ids, zeros, outs, idv, bufs,
             tables, *sems)

    return functools.partial(
        pl.kernel,
        out_type=[jax.ShapeDtypeStruct((NC, NS, rs, B), F32)] * nv,
        mesh=_mesh,
        compiler_params=_sc_params,
        scratch_types=(
            [pltpu.VMEM((2, vb, hs, CHUNK // hs), jnp.int32)]
            + [pltpu.VMEM((2, vb, hs, CHUNK // hs, B), F32)] * nv
            + [pltpu.VMEM_SHARED((nseg, B), F32)] * nv
            + [pltpu.SemaphoreType.DMA] * 4
        ),
    )(wrapped)


CPT_H = CPT // 2
_scatter_chk = _scatter_wrap(S_CHK, 2, 7, 2, CPT_H, _scatter_body)
_scatter_var_raw = _scatter_wrap(S_VAR, 1, 7, 2, CPT_H, _scatter_body)


def _scatter_var(vals, ids, zeros):
    (out,) = _scatter_var_raw(vals, ids, zeros)
    return out


def _gather_body(nv, vb, hs, cpt, tables, ids, outs, idv, bufs,
                 sg0, sg1, so0, so1):
    njo = cpt // vb
    w = _wid()
    gsems = (sg0, sg1)
    osems = (so0, so1)

    def load_ids(jo, par):
        pltpu.async_copy(ids.at[w, pl.ds(jo * vb, vb)], idv.at[par],
                         gsems[par])

    def ids_wait(jo, par):
        pltpu.make_async_copy(ids.at[w, pl.ds(jo * vb, vb)], idv.at[par],
                              gsems[par]).wait()

    def fire(par):
        return [pltpu.async_copy(t.at[idv.at[par, k, h]], buf.at[par, k, h],
                                 gsems[par])
                for t, buf in zip(tables, bufs)
                for k in range(vb) for h in range(hs)]

    def out_wait(jo, par):
        for buf, o in zip(bufs, outs):
            pltpu.make_async_copy(buf.at[par], o.at[w, jo], osems[par]).wait()

    def out_fire(jo, par):
        for buf, o in zip(bufs, outs):
            pltpu.async_copy(buf.at[par], o.at[w, jo], osems[par])

    load_ids(0, 0)
    load_ids(1, 1)

    def body(jo2, _):
        jo = jo2 * 2

        @pl.when(jo2 >= 1)
        def _():
            out_wait(jo - 2, 0)

        ids_wait(jo, 0)
        d0 = fire(0)

        @pl.when(jo2 >= 1)
        def _():
            out_wait(jo - 1, 1)

        ids_wait(jo + 1, 1)
        d1 = fire(1)
        for d in d0:
            d.wait()
        out_fire(jo, 0)

        @pl.when(jo + 2 < njo)
        def _():
            load_ids(jo + 2, 0)

        for d in d1:
            d.wait()
        out_fire(jo + 1, 1)

        @pl.when(jo + 3 < njo)
        def _():
            load_ids(jo + 3, 1)

        return 0

    lax.fori_loop(0, njo // 2, body, 0)
    out_wait(njo - 2, 0)
    out_wait(njo - 1, 1)


def _gather_wrap(nv, vb, hs, cpt, body):
    def wrapped(*args):
        tables = args[:nv]
        ids = args[nv]
        outs = args[nv + 1:nv * 2 + 1]
        idv = args[nv * 2 + 1]
        bufs = args[nv * 2 + 2:nv * 3 + 2]
        sems = args[nv * 3 + 2:]
        body(nv, vb, hs, cpt, tables, ids, outs, idv, bufs, *sems)

    return functools.partial(
        pl.kernel,
        out_type=[jax.ShapeDtypeStruct((NW, cpt // vb, vb, hs, CHUNK // hs,
                                        B), F32)] * nv,
        mesh=_mesh,
        compiler_params=_sc_params,
        scratch_types=(
            [pltpu.VMEM((2, vb, hs, CHUNK // hs), jnp.int32)]
            + [pltpu.VMEM((2, vb, hs, CHUNK // hs, B), F32)] * nv
            + [pltpu.SemaphoreType.DMA] * 4
        ),
    )(wrapped)


_gather_chk = _gather_wrap(2, 7, 2, CPT_H, _gather_body)
_gather_var_raw = _gather_wrap(1, 7, 2, CPT_H, _gather_body)


def _gather_var(table, ids):
    (out,) = _gather_var_raw(table, ids)
    return out


XR1 = EPAD * B // 128
XRH = XR1 // 2
_RBX = 3584
_EW_GRID = XRH // _RBX
_x1_spec = pl.BlockSpec((_RBX, 128), lambda i: (i, 0))


def _lognegs(v2c):
    t = jnp.tanh(v2c * 0.5)
    mag = jnp.clip(jnp.abs(t), 1e-7, 0.999999)
    lm = jnp.log(mag)
    ng = jnp.where(t < 0.0, 1.0, 0.0).astype(F32)
    return lm, ng


def _e1_body(v2c_ref, lm_ref, ng_ref):
    lm, ng = _lognegs(v2c_ref[...])
    lm_ref[...] = lm
    ng_ref[...] = ng


def _e1(v2c):
    return pl.pallas_call(
        _e1_body,
        grid=(_EW_GRID,),
        in_specs=[_x1_spec],
        out_specs=[_x1_spec, _x1_spec],
        out_shape=[jax.ShapeDtypeStruct((XRH, 128), F32)] * 2,
    )(v2c)


def _e2_body(alpha_ref, lm_ref, ng_ref, gl_ref, gn_ref, c2v_ref):
    loo_log = gl_ref[...] - lm_ref[...]
    loo_neg = gn_ref[...] - ng_ref[...]
    sign = 1.0 - 2.0 * jnp.mod(loo_neg, 2.0)
    prod = jnp.clip(sign * jnp.exp(loo_log), -0.999, 0.999)
    c2v_ref[...] = alpha_ref[0, 0] * jnp.log((1.0 + prod) / (1.0 - prod))


def _e2(alpha, lm, ng, gl, gn):
    return pl.pallas_call(
        _e2_body,
        grid=(_EW_GRID,),
        in_specs=[
            pl.BlockSpec((1, 1), lambda i: (0, 0), memory_space=pltpu.SMEM),
            _x1_spec, _x1_spec, _x1_spec, _x1_spec,
        ],
        out_specs=_x1_spec,
        out_shape=jax.ShapeDtypeStruct((XRH, 128), F32),
    )(alpha.reshape(1, 1), lm, ng, gl, gn)


def _e13_body(ch_ref, g_ref, c2v_ref, lm_ref, ng_ref):
    v2c = ch_ref[...] + g_ref[...] - c2v_ref[...]
    lm, ng = _lognegs(v2c)
    lm_ref[...] = lm
    ng_ref[...] = ng


def _e13(ch, g, c2v):
    return pl.pallas_call(
        _e13_body,
        grid=(_EW_GRID,),
        in_specs=[_x1_spec] * 3,
        out_specs=[_x1_spec] * 2,
        out_shape=[jax.ShapeDtypeStruct((XRH, 128), F32)] * 2,
    )(ch, g, c2v)


def _combine4x2_body(a0, a1, b0, b1, oa, ob):
    oa[...] = a0[0] + a0[1] + a1[0] + a1[1]
    ob[...] = b0[0] + b0[1] + b1[0] + b1[1]


def _combine4x2(pa, pb, nseg):
    rows = nseg * B // 128
    args = [p.reshape(2, rows, 128) for p in (*pa, *pb)]
    return pl.pallas_call(
        _combine4x2_body,
        out_shape=[jax.ShapeDtypeStruct((rows, 128), F32)] * 2,
    )(*args)


def _combine4_body(p0, p1, out_ref):
    out_ref[...] = p0[0] + p0[1] + p1[0] + p1[1]


def _combine4(p0, p1, nseg):
    rows = nseg * B // 128
    return pl.pallas_call(
        _combine4_body,
        out_shape=jax.ShapeDtypeStruct((rows, 128), F32),
    )(p0.reshape(2, rows, 128), p1.reshape(2, rows, 128))


def _final_body(llr_ref, p0, p1, out_ref):
    out_ref[...] = llr_ref[...] + p0[0] + p0[1] + p1[0] + p1[1]


def _final(llr_flat, pv0, pv1):
    rows = S_VAR * B // 128
    return pl.pallas_call(
        _final_body,
        out_shape=jax.ShapeDtypeStruct((rows, 128), F32),
    )(llr_flat, pv0.reshape(2, rows, 128), pv1.reshape(2, rows, 128))


def kernel(channel_llrs, edge_index, alpha):
    ids32 = edge_index.astype(jnp.int32)
    pad = EPAD - NE
    eph = EPAD // 2

    def half_ids(row, fill):
        full = jnp.concatenate([row, jnp.full((pad,), fill, jnp.int32)])
        return [full[h * eph:(h + 1) * eph].reshape(NW, CPT_H, 2, CHUNK // 2)
                for h in range(2)]

    var_ids = half_ids(ids32[0], NV)
    chk_ids = half_ids(ids32[1], NCK)

    llr_tab = jnp.pad(channel_llrs.astype(F32).T, ((0, S_VAR - NV), (0, 0)))
    llr_flat = llr_tab.reshape(S_VAR * B // 128, 128)
    z_chk = jnp.zeros((NS, S_CHK // NS, B), F32)
    z_var = jnp.zeros((NS, S_VAR // NS, B), F32)

    def rows5(flat, vb=7):
        return flat.reshape(NW, CPT_H // vb, vb, 2, CHUNK // 2, B)

    def flat2(x):
        return x.reshape(XRH, 128)

    alpha_s = alpha.astype(F32)
    ch = [flat2(_gather_var(llr_tab, var_ids[h])) for h in range(2)]
    lm = [None, None]
    ng = [None, None]
    for h in range(2):
        lm[h], ng[h] = _e1(ch[h])

    p_var = None
    for it in range(ITERS):
        p_chk = [_scatter_chk(rows5(lm[h]), rows5(ng[h]), chk_ids[h], z_chk)
                 for h in range(2)]
        tab_log, tab_neg = _combine4x2(
            (p_chk[0][0], p_chk[1][0]), (p_chk[0][1], p_chk[1][1]), S_CHK)
        tab_log = tab_log.reshape(S_CHK, B)
        tab_neg = tab_neg.reshape(S_CHK, B)
        c2v = [None, None]
        for h in range(2):
            g_log, g_neg = _gather_chk(tab_log, tab_neg, chk_ids[h])
            c2v[h] = _e2(alpha_s, lm[h], ng[h], flat2(g_log), flat2(g_neg))
        p_var = [_scatter_var(rows5(c2v[h]), var_ids[h], z_var)
                 for h in range(2)]
        if it < ITERS - 1:
            tab_var = _combine4(p_var[0], p_var[1], S_VAR).reshape(S_VAR, B)
            for h in range(2):
                g_c2v = flat2(_gather_var(tab_var, var_ids[h]))
                lm[h], ng[h] = _e13(ch[h], g_c2v, c2v[h])

    final = _final(llr_flat, p_var[0], p_var[1]).reshape(S_VAR, B)
    return final[:NV].T


# speedup vs baseline: 2.9321x; 1.0022x over previous
import functools

import jax
import jax.numpy as jnp
from jax import lax
from jax.experimental import pallas as pl
from jax.experimental.pallas import tpu as pltpu
from jax.experimental.pallas import tpu_sc as plsc

F32 = jnp.float32

NV = 50000
NCK = 25000
NE = 800000
B = 16
ITERS = 5

NC = 2
NS = 16
NW = NC * NS
CHUNK = 128
CPT = 196
EPAD = NW * CPT * CHUNK
VB = 14
VB_C = 7

S_CHK = 25024
S_VAR = 50048

_mesh = plsc.VectorSubcoreMesh(
    core_axis_name="c", subcore_axis_name="s", num_cores=NC, num_subcores=NS)
_sc_params = pltpu.CompilerParams(use_tc_tiling_on_sc=False,
                                 has_side_effects=False)


def _wid():
    return lax.axis_index("s") * NC + lax.axis_index("c")


def _scatter_body(nseg, nv, vb, hs, cpt, vals, ids, zeros, outs, idv, bufs,
                  tables, sl0, sl1, ss0, ss1):
    rs = nseg // NS
    njo = cpt // vb
    c = lax.axis_index("c")
    s = lax.axis_index("s")
    w = _wid()
    for t in tables:
        pltpu.sync_copy(zeros.at[s], t.at[pl.ds(s * rs, rs)])
    plsc.subcore_barrier()

    lsems = (sl0, sl1)
    ssems = (ss0, ss1)

    def load(jo, par):
        pltpu.async_copy(ids.at[w, pl.ds(jo * vb, vb)], idv.at[par],
                         lsems[par])
        for v, buf in zip(vals, bufs):
            pltpu.async_copy(v.at[w, jo], buf.at[par], lsems[par])

    def load_wait(jo, par):
        pltpu.make_async_copy(ids.at[w, pl.ds(jo * vb, vb)], idv.at[par],
                              lsems[par]).wait()
        for v, buf in zip(vals, bufs):
            pltpu.make_async_copy(v.at[w, jo], buf.at[par], lsems[par]).wait()

    def fire(par):
        return [pltpu.async_copy(buf.at[par, k, h], t.at[idv.at[par, k, h]],
                                 ssems[par], add=True)
                for buf, t in zip(bufs, tables)
                for k in range(vb) for h in range(hs)]

    load(0, 0)
    load(1, 1)

    def body(jo2, _):
        jo = jo2 * 2
        load_wait(jo, 0)
        d0 = fire(0)
        load_wait(jo + 1, 1)
        d1 = fire(1)
        for d in d0:
            d.wait()

        @pl.when(jo + 2 < njo)
        def _():
            load(jo + 2, 0)

        for d in d1:
            d.wait()

        @pl.when(jo + 3 < njo)
        def _():
            load(jo + 3, 1)

        return 0

    lax.fori_loop(0, njo // 2, body, 0)
    plsc.subcore_barrier()
    for t, o in zip(tables, outs):
        pltpu.sync_copy(t.at[pl.ds(s * rs, rs)], o.at[c, s])


def _scatter_wrap(nseg, nv, vb, hs, cpt, body):
    rs = nseg // NS

    def wrapped(*args):
        vals = args[:nv]
        ids, zeros = args[nv], args[nv + 1]
        outs = args[nv + 2:nv + 2 + nv]
        idv = args[nv * 2 + 2]
        bufs = args[nv * 2 + 3:nv * 3 + 3]
        tables = args[nv * 3 + 3:nv * 4 + 3]
        sems = args[nv * 4 + 3:]
        body(nseg, nv, vb, hs, cpt, vals, ids, zeros, outs, idv, bufs,
             tables, *sems)

    return functools.partial(
        pl.kernel,
        out_type=[jax.ShapeDtypeStruct((NC, NS, rs, B), F32)] * nv,
        mesh=_mesh,
        compiler_params=_sc_params,
        scratch_types=(
            [pltpu.VMEM((2, vb, hs, CHUNK // hs), jnp.int32)]
            + [pltpu.VMEM((2, vb, hs, CHUNK // hs, B), F32)] * nv
            + [pltpu.VMEM_SHARED((nseg, B), F32)] * nv
            + [pltpu.SemaphoreType.DMA] * 4
        ),
    )(wrapped)


CPT_H = CPT // 2
_scatter_chk = _scatter_wrap(S_CHK, 2, 7, 2, CPT_H, _scatter_body)
_scatter_var_raw = _scatter_wrap(S_VAR, 1, 7, 2, CPT_H, _scatter_body)


def _scatter_var(vals, ids, zeros):
    (out,) = _scatter_var_raw(vals, ids, zeros)
    return out


def _gather_body(nv, vb, hs, cpt, tables, ids, outs, idv, bufs,
                 sg0, sg1, so0, so1):
    njo = cpt // vb
    w = _wid()
    gsems = (sg0, sg1)
    osems = (so0, so1)

    def load_ids(jo, par):
        pltpu.async_copy(ids.at[w, pl.ds(jo * vb, vb)], idv.at[par],
                         gsems[par])

    def ids_wait(jo, par):
        pltpu.make_async_copy(ids.at[w, pl.ds(jo * vb, vb)], idv.at[par],
                              gsems[par]).wait()

    def fire(par):
        return [pltpu.async_copy(t.at[idv.at[par, k, h]], buf.at[par, k, h],
                                 gsems[par])
                for t, buf in zip(tables, bufs)
                for k in range(vb) for h in range(hs)]

    def out_wait(jo, par):
        for buf, o in zip(bufs, outs):
            pltpu.make_async_copy(buf.at[par], o.at[w, jo], osems[par]).wait()

    def out_fire(jo, par):
        for buf, o in zip(bufs, outs):
            pltpu.async_copy(buf.at[par], o.at[w, jo], osems[par])

    load_ids(0, 0)
    load_ids(1, 1)

    def body(jo2, _):
        jo = jo2 * 2

        @pl.when(jo2 >= 1)
        def _():
            out_wait(jo - 2, 0)

        ids_wait(jo, 0)
        d0 = fire(0)

        @pl.when(jo2 >= 1)
        def _():
            out_wait(jo - 1, 1)

        ids_wait(jo + 1, 1)
        d1 = fire(1)
        for d in d0:
            d.wait()
        out_fire(jo, 0)

        @pl.when(jo + 2 < njo)
        def _():
            load_ids(jo + 2, 0)

        for d in d1:
            d.wait()
        out_fire(jo + 1, 1)

        @pl.when(jo + 3 < njo)
        def _():
            load_ids(jo + 3, 1)

        return 0

    lax.fori_loop(0, njo // 2, body, 0)
    out_wait(njo - 2, 0)
    out_wait(njo - 1, 1)


def _gather_wrap(nv, vb, hs, cpt, body):
    def wrapped(*args):
        tables = args[:nv]
        ids = args[nv]
        outs = args[nv + 1:nv * 2 + 1]
        idv = args[nv * 2 + 1]
        bufs = args[nv * 2 + 2:nv * 3 + 2]
        sems = args[nv * 3 + 2:]
        body(nv, vb, hs, cpt, tables, ids, outs, idv, bufs, *sems)

    return functools.partial(
        pl.kernel,
        out_type=[jax.ShapeDtypeStruct((NW, cpt // vb, vb, hs, CHUNK // hs,
                                        B), F32)] * nv,
        mesh=_mesh,
        compiler_params=_sc_params,
        scratch_types=(
            [pltpu.VMEM((2, vb, hs, CHUNK // hs), jnp.int32)]
            + [pltpu.VMEM((2, vb, hs, CHUNK // hs, B), F32)] * nv
            + [pltpu.SemaphoreType.DMA] * 4
        ),
    )(wrapped)


_gather_chk = _gather_wrap(2, 7, 2, CPT_H, _gather_body)
_gather_var_raw = _gather_wrap(1, 7, 2, CPT_H, _gather_body)


def _gather_var(table, ids):
    (out,) = _gather_var_raw(table, ids)
    return out


XR1 = EPAD * B // 128
XRH = XR1 // 2
_RBX = 3584
_EW_GRID = XRH // _RBX
_x1_spec = pl.BlockSpec((_RBX, 128), lambda i: (i, 0))


def _lognegs(v2c):
    t = jnp.tanh(v2c * 0.5)
    mag = jnp.clip(jnp.abs(t), 1e-7, 0.999999)
    lm = jnp.log(mag)
    ng = jnp.where(t < 0.0, 1.0, 0.0).astype(F32)
    return lm, ng


def _e1_body(v2c_ref, lm_ref, ng_ref):
    lm, ng = _lognegs(v2c_ref[...])
    lm_ref[...] = lm
    ng_ref[...] = ng


def _e1(v2c):
    return pl.pallas_call(
        _e1_body,
        grid=(_EW_GRID,),
        in_specs=[_x1_spec],
        out_specs=[_x1_spec, _x1_spec],
        out_shape=[jax.ShapeDtypeStruct((XRH, 128), F32)] * 2,
    )(v2c)


def _e2_body(alpha_ref, lm_ref, ng_ref, gl_ref, gn_ref, c2v_ref):
    loo_log = gl_ref[...] - lm_ref[...]
    loo_neg = gn_ref[...] - ng_ref[...]
    sign = 1.0 - 2.0 * jnp.mod(loo_neg, 2.0)
    prod = jnp.clip(sign * jnp.exp(loo_log), -0.999, 0.999)
    c2v_ref[...] = alpha_ref[0, 0] * jnp.log((1.0 + prod) / (1.0 - prod))


def _e2(alpha, lm, ng, gl, gn):
    return pl.pallas_call(
        _e2_body,
        grid=(_EW_GRID,),
        in_specs=[
            pl.BlockSpec((1, 1), lambda i: (0, 0), memory_space=pltpu.SMEM),
            _x1_spec, _x1_spec, _x1_spec, _x1_spec,
        ],
        out_specs=_x1_spec,
        out_shape=jax.ShapeDtypeStruct((XRH, 128), F32),
    )(alpha.reshape(1, 1), lm, ng, gl, gn)


def _e13_body(ch_ref, g_ref, c2v_ref, lm_ref, ng_ref):
    v2c = ch_ref[...] + g_ref[...] - c2v_ref[...]
    lm, ng = _lognegs(v2c)
    lm_ref[...] = lm
    ng_ref[...] = ng


def _e13(ch, g, c2v):
    return pl.pallas_call(
        _e13_body,
        grid=(_EW_GRID,),
        in_specs=[_x1_spec] * 3,
        out_specs=[_x1_spec] * 2,
        out_shape=[jax.ShapeDtypeStruct((XRH, 128), F32)] * 2,
    )(ch, g, c2v)


def _combine4x2_body(a0, a1, b0, b1, oa, ob):
    oa[...] = a0[0] + a0[1] + a1[0] + a1[1]
    ob[...] = b0[0] + b0[1] + b1[0] + b1[1]


def _combine4x2(pa, pb, nseg):
    rows = nseg * B // 128
    args = [p.reshape(2, rows, 128) for p in (*pa, *pb)]
    return pl.pallas_call(
        _combine4x2_body,
        out_shape=[jax.ShapeDtypeStruct((rows, 128), F32)] * 2,
    )(*args)


def _combine4_body(p0, p1, out_ref):
    out_ref[...] = p0[0] + p0[1] + p1[0] + p1[1]


def _combine4(p0, p1, nseg):
    rows = nseg * B // 128
    return pl.pallas_call(
        _combine4_body,
        out_shape=jax.ShapeDtypeStruct((rows, 128), F32),
    )(p0.reshape(2, rows, 128), p1.reshape(2, rows, 128))


def _final_body(llr_ref, p0, p1, out_ref):
    out_ref[...] = llr_ref[...] + p0[0] + p0[1] + p1[0] + p1[1]


def _final(llr_flat, pv0, pv1):
    rows = S_VAR * B // 128
    return pl.pallas_call(
        _final_body,
        out_shape=jax.ShapeDtypeStruct((rows, 128), F32),
    )(llr_flat, pv0.reshape(2, rows, 128), pv1.reshape(2, rows, 128))


def kernel(channel_llrs, edge_index, alpha):
    ids32 = edge_index.astype(jnp.int32)
    pad = EPAD - NE
    eph = EPAD // 2

    def half_ids(row, fill):
        full = jnp.concatenate([row, jnp.full((pad,), fill, jnp.int32)])
        return [full[h * eph:(h + 1) * eph].reshape(NW, CPT_H, 2, CHUNK // 2)
                for h in range(2)]

    var_ids = half_ids(ids32[0], NV)
    chk_ids = half_ids(ids32[1], NCK)

    llr_tab = jnp.pad(channel_llrs.astype(F32).T, ((0, S_VAR - NV), (0, 0)))
    llr_flat = llr_tab.reshape(S_VAR * B // 128, 128)
    z_chk = jnp.zeros((NS, S_CHK // NS, B), F32)
    z_var = jnp.zeros((NS, S_VAR // NS, B), F32)

    def rows5(flat, vb=7):
        return flat.reshape(NW, CPT_H // vb, vb, 2, CHUNK // 2, B)

    def flat2(x):
        return x.reshape(XRH, 128)

    alpha_s = alpha.astype(F32)
    ch = [flat2(_gather_var(llr_tab, var_ids[h])) for h in range(2)]
    lm = [None, None]
    ng = [None, None]
    for h in range(2):
        lm[h], ng[h] = _e1(ch[h])

    p_var = None
    for it in range(ITERS):
        p_chk = [_scatter_chk(rows5(lm[h]), rows5(ng[h]), chk_ids[h], z_chk)
                 for h in range(2)]
        tab_log, tab_neg = _combine4x2(
            (p_chk[0][0], p_chk[1][0]), (p_chk[0][1], p_chk[1][1]), S_CHK)
        tab_log = tab_log.reshape(S_CHK, B)
        tab_neg = tab_neg.reshape(S_CHK, B)
        c2v = [None, None]
        for h in range(2):
            g_log, g_neg = _gather_chk(tab_log, tab_neg, chk_ids[h])
            c2v[h] = _e2(alpha_s, lm[h], ng[h], flat2(g_log), flat2(g_neg))
        p_var = [_scatter_var(rows5(c2v[h]), var_ids[h], z_var)
                 for h in range(2)]
        if it < ITERS - 1:
            tab_var = _combine4(p_var[0], p_var[1], S_VAR).reshape(S_VAR, B)
            for h in range(2):
                g_c2v = flat2(_gather_var(tab_var, var_ids[h]))
                lm[h], ng[h] = _e13(ch[h], g_c2v, c2v[h])

    final = _final(llr_flat, p_var[0], p_var[1]).reshape(S_VAR, B)
    return final[:NV].T
